# baseline (speedup 1.0000x reference)
import functools

import jax
import jax.numpy as jnp
from jax import lax
from jax.experimental import pallas as pl
from jax.experimental.pallas import tpu as pltpu
from jax.experimental.pallas import tpu_sc as plsc

T = 2048
SI = 1024
D = 768
HEADS = 12
DH = 64
E = 8
HID = 3072
BS = 256
MAXBLK = 24
MAXROWS = MAXBLK * BS
NW = 32
TPW = T // NW


def _proj_body(x_ref, img_ref, g_ref, wq_ref, wkv_ref, q_ref, kv_ref):
    x = x_ref[...]
    mu = jnp.mean(x, axis=-1, keepdims=True)
    var = jnp.mean((x - mu) ** 2, axis=-1, keepdims=True)
    xn = (x - mu) / jnp.sqrt(var + 1e-5) * g_ref[...]
    scale = DH ** -0.5
    q_ref[...] = jnp.dot(xn.astype(jnp.bfloat16), wq_ref[...],
                         preferred_element_type=jnp.float32) * scale
    kv_ref[...] = jnp.dot(img_ref[...], wkv_ref[...],
                          preferred_element_type=jnp.float32)


def _proj(x, img, g, Wq, Wkv):
    return pl.pallas_call(
        _proj_body,
        out_shape=(jax.ShapeDtypeStruct((T, D), jnp.float32),
                   jax.ShapeDtypeStruct((SI, 2 * D), jnp.float32)),
    )(x, img, g, Wq, Wkv)


def _attn_body(q_ref, k_ref, v_ref, nk_ref, nv_ref, o_ref):
    for j in range(2):
        qf = q_ref[:, j * DH:(j + 1) * DH]
        q = qf.astype(jnp.bfloat16)
        k = k_ref[:, j * DH:(j + 1) * DH].astype(jnp.bfloat16)
        v = v_ref[:, j * DH:(j + 1) * DH].astype(jnp.bfloat16)
        sim = lax.dot_general(q, k, (((1,), (1,)), ((), ())),
                              preferred_element_type=jnp.float32)
        snull = lax.dot_general(qf, nk_ref[...], (((1,), (1,)), ((), ())),
                                preferred_element_type=jnp.float32)
        m = jnp.maximum(jnp.max(sim, axis=-1, keepdims=True), snull)
        eimg = jnp.exp(sim - m)
        enull = jnp.exp(snull - m)
        denom = enull + jnp.sum(eimg, axis=-1, keepdims=True)
        attn = (eimg / denom).astype(jnp.bfloat16)
        anull = enull / denom
        out = (jnp.dot(attn, v, preferred_element_type=jnp.float32)
               + anull * nv_ref[...])
        o_ref[:, j * DH:(j + 1) * DH] = out


def _attention(q, kv, nk, nv):
    return pl.pallas_call(
        _attn_body,
        grid=(HEADS // 2,),
        in_specs=[
            pl.BlockSpec((T, 2 * DH), lambda h: (0, h)),
            pl.BlockSpec((SI, 2 * DH), lambda h: (0, h)),
            pl.BlockSpec((SI, 2 * DH), lambda h: (0, h + HEADS // 2)),
            pl.BlockSpec((1, DH), lambda h: (0, 0)),
            pl.BlockSpec((1, DH), lambda h: (0, 0)),
        ],
        out_specs=pl.BlockSpec((T, 2 * DH), lambda h: (0, h)),
        out_shape=jax.ShapeDtypeStruct((T, HEADS * DH), jnp.float32),
    )(q, kv, kv, nk, nv)


def _post_body(ao_ref, res_ref, wo_ref, go_ref, gw_ref,
               act_ref, d0_ref, d1_ref, w0_ref, w1_ref, eob_ref, nb_ref):
    y = jnp.dot(ao_ref[...].astype(jnp.bfloat16), wo_ref[...],
                preferred_element_type=jnp.float32)
    mu = jnp.mean(y, axis=-1, keepdims=True)
    var = jnp.mean((y - mu) ** 2, axis=-1, keepdims=True)
    attended = (y - mu) / jnp.sqrt(var + 1e-5) * go_ref[...]
    act = jnp.tanh(attended) + res_ref[...]
    act_ref[...] = act

    logits = jnp.dot(act.astype(jnp.bfloat16), gw_ref[...],
                     preferred_element_type=jnp.float32)
    lm = jnp.max(logits, axis=-1, keepdims=True)
    ex = jnp.exp(logits - lm)
    gates = ex / jnp.sum(ex, axis=-1, keepdims=True)

    lane = lax.broadcasted_iota(jnp.int32, (T, E), 1)
    v0 = jnp.max(gates, axis=-1, keepdims=True)
    i0 = jnp.min(jnp.where(gates == v0, lane, E), axis=-1, keepdims=True)
    oh0 = lane == i0
    gates1 = jnp.where(oh0, -1.0, gates)
    v1 = jnp.max(gates1, axis=-1, keepdims=True)
    i1 = jnp.min(jnp.where(gates1 == v1, lane, E), axis=-1, keepdims=True)
    oh1 = lane == i1
    s = v0 + v1 + 1e-9
    w0_ref[...] = v0 / s
    w1_ref[...] = v1 / s

    sel = oh0.astype(jnp.float32) + oh1.astype(jnp.float32)
    cum = sel
    sft = 1
    while sft < T:
        top = jnp.zeros((sft, E), jnp.float32)
        cum = cum + jnp.concatenate([top, cum[:T - sft]], axis=0)
        sft *= 2
    counts = cum[T - 1:T, :]
    aligned = jnp.ceil(counts / BS) * BS
    ec = lax.broadcasted_iota(jnp.int32, (E, E), 0)
    er = lax.broadcasted_iota(jnp.int32, (E, E), 1)
    tri = (ec < er).astype(jnp.float32)
    off = jnp.dot(aligned, tri, preferred_element_type=jnp.float32)
    end = off + aligned

    pos = off + cum - 1.0
    d0 = jnp.sum(jnp.where(oh0, pos, 0.0), axis=-1, keepdims=True)
    d1 = jnp.sum(jnp.where(oh1, pos, 0.0), axis=-1, keepdims=True)
    d0_ref[...] = d0.astype(jnp.int32)
    d1_ref[...] = d1.astype(jnp.int32)

    bstart = lax.broadcasted_iota(jnp.int32, (1, 128), 1) * BS
    endc = jnp.reshape(end, (E, 1)).astype(jnp.int32)
    eob = jnp.sum((bstart >= endc).astype(jnp.int32), axis=0, keepdims=True)
    eob_ref[...] = jnp.minimum(eob, E - 1)
    nb_ref[...] = (jnp.sum(aligned) / BS).astype(jnp.int32).reshape(1, 1)


def _post(ao, res, Wo, go, gw):
    return pl.pallas_call(
        _post_body,
        out_shape=(
            jax.ShapeDtypeStruct((T, D), jnp.float32),
            jax.ShapeDtypeStruct((T, 1), jnp.int32),
            jax.ShapeDtypeStruct((T, 1), jnp.int32),
            jax.ShapeDtypeStruct((T, 1), jnp.float32),
            jax.ShapeDtypeStruct((T, 1), jnp.float32),
            jax.ShapeDtypeStruct((1, 128), jnp.int32),
            jax.ShapeDtypeStruct((1, 1), jnp.int32),
        ),
    )(ao, res, Wo, go, gw)


def _sc_scatter(act, d0, d1):
    mesh = plsc.VectorSubcoreMesh(core_axis_name="c", subcore_axis_name="s")

    @functools.partial(
        pl.kernel, mesh=mesh,
        out_type=jax.ShapeDtypeStruct((MAXROWS, D), jnp.float32),
        scratch_types=[
            pltpu.VMEM((TPW, D), jnp.float32),
            pltpu.VMEM((TPW,), jnp.int32),
            pltpu.SemaphoreType.DMA,
        ],
    )
    def k(act_hbm, d0_hbm, d1_hbm, xs_hbm, rows_v, idx_v, sem):
        wid = lax.axis_index("s") * 2 + lax.axis_index("c")
        base = wid * TPW
        pltpu.sync_copy(act_hbm.at[pl.ds(base, TPW)], rows_v)
        pltpu.sync_copy(d0_hbm.at[pl.ds(base, TPW)], idx_v)
        pltpu.async_copy(rows_v, xs_hbm.at[idx_v], sem).wait()
        pltpu.sync_copy(d1_hbm.at[pl.ds(base, TPW)], idx_v)
        pltpu.async_copy(rows_v, xs_hbm.at[idx_v], sem).wait()

    return k(act, d0, d1)


def _sc_gather(y, d0, d1):
    mesh = plsc.VectorSubcoreMesh(core_axis_name="c", subcore_axis_name="s")

    @functools.partial(
        pl.kernel, mesh=mesh,
        out_type=(jax.ShapeDtypeStruct((T, D), jnp.float32),
                  jax.ShapeDtypeStruct((T, D), jnp.float32)),
        scratch_types=[
            pltpu.VMEM((TPW, D), jnp.float32),
            pltpu.VMEM((TPW,), jnp.int32),
            pltpu.SemaphoreType.DMA,
        ],
    )
    def k(y_hbm, d0_hbm, d1_hbm, g0_hbm, g1_hbm, rows_v, idx_v, sem):
        wid = lax.axis_index("s") * 2 + lax.axis_index("c")
        base = wid * TPW
        pltpu.sync_copy(d0_hbm.at[pl.ds(base, TPW)], idx_v)
        pltpu.async_copy(y_hbm.at[idx_v], rows_v, sem).wait()
        pltpu.sync_copy(rows_v, g0_hbm.at[pl.ds(base, TPW)])
        pltpu.sync_copy(d1_hbm.at[pl.ds(base, TPW)], idx_v)
        pltpu.async_copy(y_hbm.at[idx_v], rows_v, sem).wait()
        pltpu.sync_copy(rows_v, g1_hbm.at[pl.ds(base, TPW)])

    return k(y, d0, d1)


def _ffn_body(eob_ref, nb_ref, x_ref, w1_ref, w2_ref, o_ref):
    g = pl.program_id(0)

    @pl.when(g < nb_ref[0])
    def _():
        hmid = jax.nn.gelu(jnp.dot(x_ref[...].astype(jnp.bfloat16),
                                   w1_ref[0],
                                   preferred_element_type=jnp.float32))
        o_ref[...] = jnp.dot(hmid.astype(jnp.bfloat16), w2_ref[0],
                             preferred_element_type=jnp.float32)


def _moe_ffn(eob, nb, xs, W1, W2):
    spec = pltpu.PrefetchScalarGridSpec(
        num_scalar_prefetch=2,
        grid=(MAXBLK,),
        in_specs=[
            pl.BlockSpec((BS, D), lambda g, eob, nb: (g, 0)),
            pl.BlockSpec((1, D, HID), lambda g, eob, nb: (eob[g], 0, 0)),
            pl.BlockSpec((1, HID, D), lambda g, eob, nb: (eob[g], 0, 0)),
        ],
        out_specs=pl.BlockSpec((BS, D), lambda g, eob, nb: (g, 0)),
    )
    return pl.pallas_call(
        _ffn_body,
        grid_spec=spec,
        out_shape=jax.ShapeDtypeStruct((MAXROWS, D), jnp.float32),
    )(eob, nb, xs, W1, W2)


def _final_body(act_ref, g0_ref, g1_ref, w0_ref, w1_ref, o_ref):
    o_ref[...] = jnp.tanh(w0_ref[...] * g0_ref[...]
                          + w1_ref[...] * g1_ref[...] + act_ref[...])


def _final(act, g0, g1, w0, w1):
    return pl.pallas_call(
        _final_body,
        out_shape=jax.ShapeDtypeStruct((T, D), jnp.float32),
    )(act, g0, g1, w0, w1)


def kernel(text, img, ln_q_g, Wq, Wkv, null_k, null_v, Wo, ln_out_g, gate_W,
           expert_W1, expert_W2):
    x = text.reshape(T, D)
    im = img.reshape(SI, D)
    bf = jnp.bfloat16
    q, kv = _proj(x, im.astype(bf), ln_q_g.reshape(1, D), Wq.astype(bf),
                  Wkv.astype(bf))
    ao = _attention(q, kv, null_k.reshape(1, DH), null_v.reshape(1, DH))
    act, d0, d1, w0, w1, eob, nb = _post(ao, x, Wo.astype(bf),
                                         ln_out_g.reshape(1, D),
                                         gate_W.astype(bf))
    d0f = d0.reshape(T)
    d1f = d1.reshape(T)
    xs = _sc_scatter(act, d0f, d1f)
    y = _moe_ffn(eob[0, :MAXBLK], nb.reshape(1), xs, expert_W1.astype(bf),
                 expert_W2.astype(bf))
    g0, g1 = _sc_gather(y, d0f, d1f)
    out = _final(act, g0, g1, w0, w1)
    return out.reshape(1, T, D)


# speedup vs baseline: 1.0069x; 1.0069x over previous
import functools

import jax
import jax.numpy as jnp
from jax import lax
from jax.experimental import pallas as pl
from jax.experimental.pallas import tpu as pltpu
from jax.experimental.pallas import tpu_sc as plsc

T = 2048
SI = 1024
D = 768
HEADS = 12
DH = 64
E = 8
HID = 3072
BS = 256
MAXBLK = 24
MAXROWS = MAXBLK * BS
NW = 32
TPW = T // NW


def _proj_body(x_ref, img_ref, g_ref, wq_ref, wkv_ref, q_ref, kv_ref):
    x = x_ref[...]
    mu = jnp.mean(x, axis=-1, keepdims=True)
    var = jnp.mean((x - mu) ** 2, axis=-1, keepdims=True)
    xn = (x - mu) / jnp.sqrt(var + 1e-5) * g_ref[...]
    scale = DH ** -0.5
    q_ref[...] = jnp.dot(xn, wq_ref[...],
                         preferred_element_type=jnp.float32) * scale
    kv_ref[...] = jnp.dot(img_ref[...], wkv_ref[...],
                          preferred_element_type=jnp.float32)


def _proj(x, img, g, Wq, Wkv):
    return pl.pallas_call(
        _proj_body,
        out_shape=(jax.ShapeDtypeStruct((T, D), jnp.float32),
                   jax.ShapeDtypeStruct((SI, 2 * D), jnp.float32)),
    )(x, img, g, Wq, Wkv)


def _attn_body(q_ref, k_ref, v_ref, nk_ref, nv_ref, o_ref):
    for j in range(2):
        q = q_ref[:, j * DH:(j + 1) * DH]
        k = k_ref[:, j * DH:(j + 1) * DH]
        v = v_ref[:, j * DH:(j + 1) * DH]
        sim = lax.dot_general(q, k, (((1,), (1,)), ((), ())),
                              preferred_element_type=jnp.float32)
        snull = lax.dot_general(q, nk_ref[...], (((1,), (1,)), ((), ())),
                                preferred_element_type=jnp.float32)
        m = jnp.maximum(jnp.max(sim, axis=-1, keepdims=True), snull)
        eimg = jnp.exp(sim - m)
        enull = jnp.exp(snull - m)
        denom = enull + jnp.sum(eimg, axis=-1, keepdims=True)
        attn = eimg / denom
        anull = enull / denom
        out = (jnp.dot(attn, v, preferred_element_type=jnp.float32)
               + anull * nv_ref[...])
        o_ref[:, j * DH:(j + 1) * DH] = out


def _attention(q, kv, nk, nv):
    return pl.pallas_call(
        _attn_body,
        grid=(HEADS // 2,),
        in_specs=[
            pl.BlockSpec((T, 2 * DH), lambda h: (0, h)),
            pl.BlockSpec((SI, 2 * DH), lambda h: (0, h)),
            pl.BlockSpec((SI, 2 * DH), lambda h: (0, h + HEADS // 2)),
            pl.BlockSpec((1, DH), lambda h: (0, 0)),
            pl.BlockSpec((1, DH), lambda h: (0, 0)),
        ],
        out_specs=pl.BlockSpec((T, 2 * DH), lambda h: (0, h)),
        out_shape=jax.ShapeDtypeStruct((T, HEADS * DH), jnp.float32),
    )(q, kv, kv, nk, nv)


def _post_body(ao_ref, res_ref, wo_ref, go_ref, gw_ref,
               act_ref, d0_ref, d1_ref, w0_ref, w1_ref, eob_ref, nb_ref):
    y = jnp.dot(ao_ref[...], wo_ref[...], preferred_element_type=jnp.float32)
    mu = jnp.mean(y, axis=-1, keepdims=True)
    var = jnp.mean((y - mu) ** 2, axis=-1, keepdims=True)
    attended = (y - mu) / jnp.sqrt(var + 1e-5) * go_ref[...]
    act = jnp.tanh(attended) + res_ref[...]
    act_ref[...] = act

    logits = jnp.dot(act, gw_ref[...], preferred_element_type=jnp.float32)
    lm = jnp.max(logits, axis=-1, keepdims=True)
    ex = jnp.exp(logits - lm)
    gates = ex / jnp.sum(ex, axis=-1, keepdims=True)

    lane = lax.broadcasted_iota(jnp.int32, (T, E), 1)
    v0 = jnp.max(gates, axis=-1, keepdims=True)
    i0 = jnp.min(jnp.where(gates == v0, lane, E), axis=-1, keepdims=True)
    oh0 = lane == i0
    gates1 = jnp.where(oh0, -1.0, gates)
    v1 = jnp.max(gates1, axis=-1, keepdims=True)
    i1 = jnp.min(jnp.where(gates1 == v1, lane, E), axis=-1, keepdims=True)
    oh1 = lane == i1
    s = v0 + v1 + 1e-9
    w0_ref[...] = v0 / s
    w1_ref[...] = v1 / s

    sel = oh0.astype(jnp.float32) + oh1.astype(jnp.float32)
    cum = sel
    sft = 1
    while sft < T:
        top = jnp.zeros((sft, E), jnp.float32)
        cum = cum + jnp.concatenate([top, cum[:T - sft]], axis=0)
        sft *= 2
    counts = cum[T - 1:T, :]
    aligned = jnp.ceil(counts / BS) * BS
    ec = lax.broadcasted_iota(jnp.int32, (E, E), 0)
    er = lax.broadcasted_iota(jnp.int32, (E, E), 1)
    tri = (ec < er).astype(jnp.float32)
    off = jnp.dot(aligned, tri, preferred_element_type=jnp.float32)
    end = off + aligned

    pos = off + cum - 1.0
    d0 = jnp.sum(jnp.where(oh0, pos, 0.0), axis=-1, keepdims=True)
    d1 = jnp.sum(jnp.where(oh1, pos, 0.0), axis=-1, keepdims=True)
    d0_ref[...] = d0.astype(jnp.int32)
    d1_ref[...] = d1.astype(jnp.int32)

    bstart = lax.broadcasted_iota(jnp.int32, (1, 128), 1) * BS
    endc = jnp.reshape(end, (E, 1)).astype(jnp.int32)
    eob = jnp.sum((bstart >= endc).astype(jnp.int32), axis=0, keepdims=True)
    eob_ref[...] = jnp.minimum(eob, E - 1)
    nb_ref[...] = (jnp.sum(aligned) / BS).astype(jnp.int32).reshape(1, 1)


def _post(ao, res, Wo, go, gw):
    return pl.pallas_call(
        _post_body,
        out_shape=(
            jax.ShapeDtypeStruct((T, D), jnp.float32),
            jax.ShapeDtypeStruct((T, 1), jnp.int32),
            jax.ShapeDtypeStruct((T, 1), jnp.int32),
            jax.ShapeDtypeStruct((T, 1), jnp.float32),
            jax.ShapeDtypeStruct((T, 1), jnp.float32),
            jax.ShapeDtypeStruct((1, 128), jnp.int32),
            jax.ShapeDtypeStruct((1, 1), jnp.int32),
        ),
    )(ao, res, Wo, go, gw)


def _sc_scatter(act, d0, d1):
    mesh = plsc.VectorSubcoreMesh(core_axis_name="c", subcore_axis_name="s")

    @functools.partial(
        pl.kernel, mesh=mesh,
        out_type=jax.ShapeDtypeStruct((MAXROWS, D), jnp.float32),
        scratch_types=[
            pltpu.VMEM((TPW, D), jnp.float32),
            pltpu.VMEM((TPW,), jnp.int32),
            pltpu.SemaphoreType.DMA,
        ],
    )
    def k(act_hbm, d0_hbm, d1_hbm, xs_hbm, rows_v, idx_v, sem):
        wid = lax.axis_index("s") * 2 + lax.axis_index("c")
        base = wid * TPW
        pltpu.sync_copy(act_hbm.at[pl.ds(base, TPW)], rows_v)
        pltpu.sync_copy(d0_hbm.at[pl.ds(base, TPW)], idx_v)
        pltpu.async_copy(rows_v, xs_hbm.at[idx_v], sem).wait()
        pltpu.sync_copy(d1_hbm.at[pl.ds(base, TPW)], idx_v)
        pltpu.async_copy(rows_v, xs_hbm.at[idx_v], sem).wait()

    return k(act, d0, d1)


def _sc_gather(y, d0, d1):
    mesh = plsc.VectorSubcoreMesh(core_axis_name="c", subcore_axis_name="s")

    @functools.partial(
        pl.kernel, mesh=mesh,
        out_type=(jax.ShapeDtypeStruct((T, D), jnp.float32),
                  jax.ShapeDtypeStruct((T, D), jnp.float32)),
        scratch_types=[
            pltpu.VMEM((TPW, D), jnp.float32),
            pltpu.VMEM((TPW,), jnp.int32),
            pltpu.SemaphoreType.DMA,
        ],
    )
    def k(y_hbm, d0_hbm, d1_hbm, g0_hbm, g1_hbm, rows_v, idx_v, sem):
        wid = lax.axis_index("s") * 2 + lax.axis_index("c")
        base = wid * TPW
        pltpu.sync_copy(d0_hbm.at[pl.ds(base, TPW)], idx_v)
        pltpu.async_copy(y_hbm.at[idx_v], rows_v, sem).wait()
        pltpu.sync_copy(rows_v, g0_hbm.at[pl.ds(base, TPW)])
        pltpu.sync_copy(d1_hbm.at[pl.ds(base, TPW)], idx_v)
        pltpu.async_copy(y_hbm.at[idx_v], rows_v, sem).wait()
        pltpu.sync_copy(rows_v, g1_hbm.at[pl.ds(base, TPW)])

    return k(y, d0, d1)


def _ffn_body(eob_ref, nb_ref, x_ref, w1_ref, w2_ref, o_ref):
    g = pl.program_id(0)

    @pl.when(g < nb_ref[0])
    def _():
        hmid = jax.nn.gelu(jnp.dot(x_ref[...].astype(jnp.bfloat16),
                                   w1_ref[0],
                                   preferred_element_type=jnp.float32))
        o_ref[...] = jnp.dot(hmid.astype(jnp.bfloat16), w2_ref[0],
                             preferred_element_type=jnp.float32)


def _moe_ffn(eob, nb, xs, W1, W2):
    spec = pltpu.PrefetchScalarGridSpec(
        num_scalar_prefetch=2,
        grid=(MAXBLK,),
        in_specs=[
            pl.BlockSpec((BS, D), lambda g, eob, nb: (g, 0)),
            pl.BlockSpec((1, D, HID), lambda g, eob, nb: (eob[g], 0, 0)),
            pl.BlockSpec((1, HID, D), lambda g, eob, nb: (eob[g], 0, 0)),
        ],
        out_specs=pl.BlockSpec((BS, D), lambda g, eob, nb: (g, 0)),
    )
    return pl.pallas_call(
        _ffn_body,
        grid_spec=spec,
        out_shape=jax.ShapeDtypeStruct((MAXROWS, D), jnp.float32),
    )(eob, nb, xs, W1, W2)


def _final_body(act_ref, g0_ref, g1_ref, w0_ref, w1_ref, o_ref):
    o_ref[...] = jnp.tanh(w0_ref[...] * g0_ref[...]
                          + w1_ref[...] * g1_ref[...] + act_ref[...])


def _final(act, g0, g1, w0, w1):
    return pl.pallas_call(
        _final_body,
        out_shape=jax.ShapeDtypeStruct((T, D), jnp.float32),
    )(act, g0, g1, w0, w1)


def kernel(text, img, ln_q_g, Wq, Wkv, null_k, null_v, Wo, ln_out_g, gate_W,
           expert_W1, expert_W2):
    x = text.reshape(T, D)
    im = img.reshape(SI, D)
    bf = jnp.bfloat16
    q, kv = _proj(x, im, ln_q_g.reshape(1, D), Wq, Wkv)
    ao = _attention(q, kv, null_k.reshape(1, DH), null_v.reshape(1, DH))
    act, d0, d1, w0, w1, eob, nb = _post(ao, x, Wo, ln_out_g.reshape(1, D),
                                         gate_W)
    d0f = d0.reshape(T)
    d1f = d1.reshape(T)
    xs = _sc_scatter(act, d0f, d1f)
    y = _moe_ffn(eob[0, :MAXBLK], nb.reshape(1), xs, expert_W1.astype(bf),
                 expert_W2.astype(bf))
    g0, g1 = _sc_gather(y, d0f, d1f)
    out = _final(act, g0, g1, w0, w1)
    return out.reshape(1, T, D)


# speedup vs baseline: 1.2085x; 1.2002x over previous
import functools

import jax
import jax.numpy as jnp
from jax import lax
from jax.experimental import pallas as pl
from jax.experimental.pallas import tpu as pltpu
from jax.experimental.pallas import tpu_sc as plsc

T = 2048
SI = 1024
D = 768
HEADS = 12
DH = 64
E = 8
HID = 3072
BS = 256
MAXBLK = 24
MAXROWS = MAXBLK * BS
NW = 32
TPW = T // NW


def _pa_body(x_ref, img_ref, g_ref, wq_ref, wkv_ref, nk_ref, nv_ref, o_ref,
             q_s, kv_s):
    i = pl.program_id(0)

    @pl.when(i == 0)
    def _():
        x = x_ref[...]
        mu = jnp.mean(x, axis=-1, keepdims=True)
        var = jnp.mean((x - mu) ** 2, axis=-1, keepdims=True)
        xn = (x - mu) / jnp.sqrt(var + 1e-5) * g_ref[...]
        scale = DH ** -0.5
        q_s[...] = jnp.dot(xn, wq_ref[...],
                           preferred_element_type=jnp.float32) * scale
        kv_s[...] = jnp.dot(img_ref[...], wkv_ref[...],
                            preferred_element_type=jnp.float32)

    @pl.when(i > 0)
    def _():
        h = i - 1
        qp = q_s[:, pl.ds(h * 2 * DH, 2 * DH)]
        kp = kv_s[:, pl.ds(h * 2 * DH, 2 * DH)]
        vp = kv_s[:, pl.ds(D + h * 2 * DH, 2 * DH)]
        for j in range(2):
            q = qp[:, j * DH:(j + 1) * DH]
            k = kp[:, j * DH:(j + 1) * DH]
            v = vp[:, j * DH:(j + 1) * DH]
            sim = lax.dot_general(q, k, (((1,), (1,)), ((), ())),
                                  preferred_element_type=jnp.float32)
            snull = lax.dot_general(q, nk_ref[...], (((1,), (1,)), ((), ())),
                                    preferred_element_type=jnp.float32)
            m = jnp.maximum(jnp.max(sim, axis=-1, keepdims=True), snull)
            eimg = jnp.exp(sim - m)
            enull = jnp.exp(snull - m)
            denom = enull + jnp.sum(eimg, axis=-1, keepdims=True)
            attn = eimg / denom
            anull = enull / denom
            out = (jnp.dot(attn, v, preferred_element_type=jnp.float32)
                   + anull * nv_ref[...])
            o_ref[:, j * DH:(j + 1) * DH] = out


def _proj_attention(x, img, g, Wq, Wkv, nk, nv):
    return pl.pallas_call(
        _pa_body,
        grid=(HEADS // 2 + 1,),
        in_specs=[
            pl.BlockSpec((T, D), lambda i: (0, 0)),
            pl.BlockSpec((SI, D), lambda i: (0, 0)),
            pl.BlockSpec((1, D), lambda i: (0, 0)),
            pl.BlockSpec((D, D), lambda i: (0, 0)),
            pl.BlockSpec((D, 2 * D), lambda i: (0, 0)),
            pl.BlockSpec((1, DH), lambda i: (0, 0)),
            pl.BlockSpec((1, DH), lambda i: (0, 0)),
        ],
        out_specs=pl.BlockSpec((T, 2 * DH),
                               lambda i: (0, jnp.maximum(i - 1, 0))),
        out_shape=jax.ShapeDtypeStruct((T, HEADS * DH), jnp.float32),
        scratch_shapes=[pltpu.VMEM((T, D), jnp.float32),
                        pltpu.VMEM((SI, 2 * D), jnp.float32)],
    )(x, img, g, Wq, Wkv, nk, nv)


def _post_body(ao_ref, res_ref, wo_ref, go_ref, gw_ref,
               act_ref, d0_ref, d1_ref, w0_ref, w1_ref, eob_ref, nb_ref):
    y = jnp.dot(ao_ref[...], wo_ref[...], preferred_element_type=jnp.float32)
    mu = jnp.mean(y, axis=-1, keepdims=True)
    var = jnp.mean((y - mu) ** 2, axis=-1, keepdims=True)
    attended = (y - mu) / jnp.sqrt(var + 1e-5) * go_ref[...]
    act = jnp.tanh(attended) + res_ref[...]
    act_ref[...] = act

    logits = jnp.dot(act, gw_ref[...], preferred_element_type=jnp.float32)
    lm = jnp.max(logits, axis=-1, keepdims=True)
    ex = jnp.exp(logits - lm)
    gates = ex / jnp.sum(ex, axis=-1, keepdims=True)

    lane = lax.broadcasted_iota(jnp.int32, (T, E), 1)
    v0 = jnp.max(gates, axis=-1, keepdims=True)
    i0 = jnp.min(jnp.where(gates == v0, lane, E), axis=-1, keepdims=True)
    oh0 = lane == i0
    gates1 = jnp.where(oh0, -1.0, gates)
    v1 = jnp.max(gates1, axis=-1, keepdims=True)
    i1 = jnp.min(jnp.where(gates1 == v1, lane, E), axis=-1, keepdims=True)
    oh1 = lane == i1
    s = v0 + v1 + 1e-9
    w0_ref[...] = v0 / s
    w1_ref[...] = v1 / s

    sel = oh0.astype(jnp.float32) + oh1.astype(jnp.float32)
    cum = sel
    sft = 1
    while sft < T:
        top = jnp.zeros((sft, E), jnp.float32)
        cum = cum + jnp.concatenate([top, cum[:T - sft]], axis=0)
        sft *= 2
    counts = cum[T - 1:T, :]
    aligned = jnp.ceil(counts / BS) * BS
    ec = lax.broadcasted_iota(jnp.int32, (E, E), 0)
    er = lax.broadcasted_iota(jnp.int32, (E, E), 1)
    tri = (ec < er).astype(jnp.float32)
    off = jnp.dot(aligned, tri, preferred_element_type=jnp.float32)
    end = off + aligned

    pos = off + cum - 1.0
    d0 = jnp.sum(jnp.where(oh0, pos, 0.0), axis=-1, keepdims=True)
    d1 = jnp.sum(jnp.where(oh1, pos, 0.0), axis=-1, keepdims=True)
    d0_ref[...] = d0.astype(jnp.int32)
    d1_ref[...] = d1.astype(jnp.int32)

    bstart = lax.broadcasted_iota(jnp.int32, (1, 128), 1) * BS
    endc = jnp.reshape(end, (E, 1)).astype(jnp.int32)
    eob = jnp.sum((bstart >= endc).astype(jnp.int32), axis=0, keepdims=True)
    eob_ref[...] = jnp.minimum(eob, E - 1)
    nb_ref[...] = (jnp.sum(aligned) / BS).astype(jnp.int32).reshape(1, 1)


def _post(ao, res, Wo, go, gw):
    return pl.pallas_call(
        _post_body,
        out_shape=(
            jax.ShapeDtypeStruct((T, D), jnp.float32),
            jax.ShapeDtypeStruct((T, 1), jnp.int32),
            jax.ShapeDtypeStruct((T, 1), jnp.int32),
            jax.ShapeDtypeStruct((T, 1), jnp.float32),
            jax.ShapeDtypeStruct((T, 1), jnp.float32),
            jax.ShapeDtypeStruct((1, 128), jnp.int32),
            jax.ShapeDtypeStruct((1, 1), jnp.int32),
        ),
    )(ao, res, Wo, go, gw)


def _sc_scatter(act, d0, d1):
    mesh = plsc.VectorSubcoreMesh(core_axis_name="c", subcore_axis_name="s")

    @functools.partial(
        pl.kernel, mesh=mesh,
        out_type=jax.ShapeDtypeStruct((MAXROWS, D), jnp.float32),
        scratch_types=[
            pltpu.VMEM((TPW, D), jnp.float32),
            pltpu.VMEM((TPW,), jnp.int32),
            pltpu.SemaphoreType.DMA,
        ],
    )
    def k(act_hbm, d0_hbm, d1_hbm, xs_hbm, rows_v, idx_v, sem):
        wid = lax.axis_index("s") * 2 + lax.axis_index("c")
        base = wid * TPW
        pltpu.sync_copy(act_hbm.at[pl.ds(base, TPW)], rows_v)
        pltpu.sync_copy(d0_hbm.at[pl.ds(base, TPW)], idx_v)
        pltpu.async_copy(rows_v, xs_hbm.at[idx_v], sem).wait()
        pltpu.sync_copy(d1_hbm.at[pl.ds(base, TPW)], idx_v)
        pltpu.async_copy(rows_v, xs_hbm.at[idx_v], sem).wait()

    return k(act, d0, d1)


def _sc_gather(y, d0, d1):
    mesh = plsc.VectorSubcoreMesh(core_axis_name="c", subcore_axis_name="s")

    @functools.partial(
        pl.kernel, mesh=mesh,
        out_type=(jax.ShapeDtypeStruct((T, D), jnp.float32),
                  jax.ShapeDtypeStruct((T, D), jnp.float32)),
        scratch_types=[
            pltpu.VMEM((TPW, D), jnp.float32),
            pltpu.VMEM((TPW,), jnp.int32),
            pltpu.SemaphoreType.DMA,
        ],
    )
    def k(y_hbm, d0_hbm, d1_hbm, g0_hbm, g1_hbm, rows_v, idx_v, sem):
        wid = lax.axis_index("s") * 2 + lax.axis_index("c")
        base = wid * TPW
        pltpu.sync_copy(d0_hbm.at[pl.ds(base, TPW)], idx_v)
        pltpu.async_copy(y_hbm.at[idx_v], rows_v, sem).wait()
        pltpu.sync_copy(rows_v, g0_hbm.at[pl.ds(base, TPW)])
        pltpu.sync_copy(d1_hbm.at[pl.ds(base, TPW)], idx_v)
        pltpu.async_copy(y_hbm.at[idx_v], rows_v, sem).wait()
        pltpu.sync_copy(rows_v, g1_hbm.at[pl.ds(base, TPW)])

    return k(y, d0, d1)


def _ffn_body(eob_ref, nb_ref, x_ref, w1_ref, w2_ref, o_ref):
    g = pl.program_id(0)

    @pl.when(g < nb_ref[0])
    def _():
        hmid = jax.nn.gelu(jnp.dot(x_ref[...], w1_ref[0],
                                   preferred_element_type=jnp.float32))
        o_ref[...] = jnp.dot(hmid, w2_ref[0],
                             preferred_element_type=jnp.float32)


def _moe_ffn(eob, nb, xs, W1, W2):
    spec = pltpu.PrefetchScalarGridSpec(
        num_scalar_prefetch=2,
        grid=(MAXBLK,),
        in_specs=[
            pl.BlockSpec((BS, D), lambda g, eob, nb: (g, 0)),
            pl.BlockSpec((1, D, HID), lambda g, eob, nb: (eob[g], 0, 0)),
            pl.BlockSpec((1, HID, D), lambda g, eob, nb: (eob[g], 0, 0)),
        ],
        out_specs=pl.BlockSpec((BS, D), lambda g, eob, nb: (g, 0)),
    )
    return pl.pallas_call(
        _ffn_body,
        grid_spec=spec,
        out_shape=jax.ShapeDtypeStruct((MAXROWS, D), jnp.float32),
    )(eob, nb, xs, W1, W2)


def _final_body(act_ref, g0_ref, g1_ref, w0_ref, w1_ref, o_ref):
    o_ref[...] = jnp.tanh(w0_ref[...] * g0_ref[...]
                          + w1_ref[...] * g1_ref[...] + act_ref[...])


def _final(act, g0, g1, w0, w1):
    return pl.pallas_call(
        _final_body,
        out_shape=jax.ShapeDtypeStruct((T, D), jnp.float32),
    )(act, g0, g1, w0, w1)


def kernel(text, img, ln_q_g, Wq, Wkv, null_k, null_v, Wo, ln_out_g, gate_W,
           expert_W1, expert_W2):
    x = text.reshape(T, D)
    im = img.reshape(SI, D)
    ao = _proj_attention(x, im, ln_q_g.reshape(1, D), Wq, Wkv,
                         null_k.reshape(1, DH), null_v.reshape(1, DH))
    act, d0, d1, w0, w1, eob, nb = _post(ao, x, Wo, ln_out_g.reshape(1, D),
                                         gate_W)
    d0f = d0.reshape(T)
    d1f = d1.reshape(T)
    xs = _sc_scatter(act, d0f, d1f)
    y = _moe_ffn(eob[0, :MAXBLK], nb.reshape(1), xs, expert_W1, expert_W2)
    g0, g1 = _sc_gather(y, d0f, d1f)
    out = _final(act, g0, g1, w0, w1)
    return out.reshape(1, T, D)


# speedup vs baseline: 1.2334x; 1.0207x over previous
import functools

import jax
import jax.numpy as jnp
from jax import lax
from jax.experimental import pallas as pl
from jax.experimental.pallas import tpu as pltpu
from jax.experimental.pallas import tpu_sc as plsc

T = 2048
SI = 1024
D = 768
HEADS = 12
DH = 64
E = 8
HID = 3072
BS = 256
MAXBLK = 24
MAXROWS = MAXBLK * BS
NW = 32
TPW = T // NW


def _pa_body(x_ref, img_ref, g_ref, wq_ref, wkv_ref, nk_ref, nv_ref,
             wo_ref, go_ref, gw_ref,
             act_ref, d0_ref, d1_ref, w0_ref, w1_ref, eob_ref, nb_ref,
             q_s, kv_s, ao_s):
    i = pl.program_id(0)

    @pl.when(i == 0)
    def _():
        x = x_ref[...]
        mu = jnp.mean(x, axis=-1, keepdims=True)
        var = jnp.mean((x - mu) ** 2, axis=-1, keepdims=True)
        xn = (x - mu) / jnp.sqrt(var + 1e-5) * g_ref[...]
        scale = DH ** -0.5
        q_s[...] = jnp.dot(xn, wq_ref[...],
                           preferred_element_type=jnp.float32) * scale
        kv_s[...] = jnp.dot(img_ref[...], wkv_ref[...],
                            preferred_element_type=jnp.float32)

    @pl.when((i > 0) & (i <= HEADS // 2))
    def _():
        h = i - 1
        qp = q_s[:, pl.ds(pl.multiple_of(h * 2 * DH, 2 * DH), 2 * DH)]
        kp = kv_s[:, pl.ds(pl.multiple_of(h * 2 * DH, 2 * DH), 2 * DH)]
        vp = kv_s[:, pl.ds(pl.multiple_of(D + h * 2 * DH, 2 * DH), 2 * DH)]
        outs = []
        for j in range(2):
            q = qp[:, j * DH:(j + 1) * DH]
            k = kp[:, j * DH:(j + 1) * DH]
            v = vp[:, j * DH:(j + 1) * DH]
            sim = lax.dot_general(q, k, (((1,), (1,)), ((), ())),
                                  preferred_element_type=jnp.float32)
            snull = lax.dot_general(q, nk_ref[...], (((1,), (1,)), ((), ())),
                                    preferred_element_type=jnp.float32)
            m = jnp.maximum(jnp.max(sim, axis=-1, keepdims=True), snull)
            eimg = jnp.exp(sim - m)
            enull = jnp.exp(snull - m)
            denom = enull + jnp.sum(eimg, axis=-1, keepdims=True)
            attn = eimg / denom
            anull = enull / denom
            outs.append(jnp.dot(attn, v, preferred_element_type=jnp.float32)
                        + anull * nv_ref[...])
        ao_s[:, pl.ds(pl.multiple_of(h * 2 * DH, 2 * DH), 2 * DH)] = (
            jnp.concatenate(outs, axis=1))

    @pl.when(i == HEADS // 2 + 1)
    def _():
        _post_compute(ao_s[...], x_ref[...], wo_ref[...], go_ref[...],
                      gw_ref[...], act_ref, d0_ref, d1_ref, w0_ref, w1_ref,
                      eob_ref, nb_ref)


def _proj_attention(x, img, g, Wq, Wkv, nk, nv, Wo, go, gw):
    cst = lambda i: (0, 0)
    return pl.pallas_call(
        _pa_body,
        grid=(HEADS // 2 + 2,),
        in_specs=[
            pl.BlockSpec((T, D), cst),
            pl.BlockSpec((SI, D), cst),
            pl.BlockSpec((1, D), cst),
            pl.BlockSpec((D, D), cst),
            pl.BlockSpec((D, 2 * D), cst),
            pl.BlockSpec((1, DH), cst),
            pl.BlockSpec((1, DH), cst),
            pl.BlockSpec((D, D), cst),
            pl.BlockSpec((1, D), cst),
            pl.BlockSpec((D, E), cst),
        ],
        out_specs=(
            pl.BlockSpec((T, D), cst),
            pl.BlockSpec((T, 1), cst),
            pl.BlockSpec((T, 1), cst),
            pl.BlockSpec((T, 1), cst),
            pl.BlockSpec((T, 1), cst),
            pl.BlockSpec((1, 128), cst),
            pl.BlockSpec((1, 1), cst),
        ),
        out_shape=(
            jax.ShapeDtypeStruct((T, D), jnp.float32),
            jax.ShapeDtypeStruct((T, 1), jnp.int32),
            jax.ShapeDtypeStruct((T, 1), jnp.int32),
            jax.ShapeDtypeStruct((T, 1), jnp.float32),
            jax.ShapeDtypeStruct((T, 1), jnp.float32),
            jax.ShapeDtypeStruct((1, 128), jnp.int32),
            jax.ShapeDtypeStruct((1, 1), jnp.int32),
        ),
        scratch_shapes=[pltpu.VMEM((T, D), jnp.float32),
                        pltpu.VMEM((SI, 2 * D), jnp.float32),
                        pltpu.VMEM((T, D), jnp.float32)],
    )(x, img, g, Wq, Wkv, nk, nv, Wo, go, gw)


def _post_compute(ao, res, wo, go, gw,
                  act_ref, d0_ref, d1_ref, w0_ref, w1_ref, eob_ref, nb_ref):
    y = jnp.dot(ao, wo, preferred_element_type=jnp.float32)
    mu = jnp.mean(y, axis=-1, keepdims=True)
    var = jnp.mean((y - mu) ** 2, axis=-1, keepdims=True)
    attended = (y - mu) / jnp.sqrt(var + 1e-5) * go
    act = jnp.tanh(attended) + res
    act_ref[...] = act

    logits = jnp.dot(act, gw, preferred_element_type=jnp.float32)
    lm = jnp.max(logits, axis=-1, keepdims=True)
    ex = jnp.exp(logits - lm)
    gates = ex / jnp.sum(ex, axis=-1, keepdims=True)

    lane = lax.broadcasted_iota(jnp.int32, (T, E), 1)
    v0 = jnp.max(gates, axis=-1, keepdims=True)
    i0 = jnp.min(jnp.where(gates == v0, lane, E), axis=-1, keepdims=True)
    oh0 = lane == i0
    gates1 = jnp.where(oh0, -1.0, gates)
    v1 = jnp.max(gates1, axis=-1, keepdims=True)
    i1 = jnp.min(jnp.where(gates1 == v1, lane, E), axis=-1, keepdims=True)
    oh1 = lane == i1
    s = v0 + v1 + 1e-9
    w0_ref[...] = v0 / s
    w1_ref[...] = v1 / s

    sel = oh0.astype(jnp.float32) + oh1.astype(jnp.float32)
    cum = sel
    sft = 1
    while sft < T:
        top = jnp.zeros((sft, E), jnp.float32)
        cum = cum + jnp.concatenate([top, cum[:T - sft]], axis=0)
        sft *= 2
    counts = cum[T - 1:T, :]
    aligned = jnp.ceil(counts / BS) * BS
    ec = lax.broadcasted_iota(jnp.int32, (E, E), 0)
    er = lax.broadcasted_iota(jnp.int32, (E, E), 1)
    tri = (ec < er).astype(jnp.float32)
    off = jnp.dot(aligned, tri, preferred_element_type=jnp.float32)
    end = off + aligned

    pos = off + cum - 1.0
    d0 = jnp.sum(jnp.where(oh0, pos, 0.0), axis=-1, keepdims=True)
    d1 = jnp.sum(jnp.where(oh1, pos, 0.0), axis=-1, keepdims=True)
    d0_ref[...] = d0.astype(jnp.int32)
    d1_ref[...] = d1.astype(jnp.int32)

    bstart = lax.broadcasted_iota(jnp.int32, (1, 128), 1) * BS
    endc = jnp.reshape(end, (E, 1)).astype(jnp.int32)
    eob = jnp.sum((bstart >= endc).astype(jnp.int32), axis=0, keepdims=True)
    eob_ref[...] = jnp.minimum(eob, E - 1)
    nb_ref[...] = (jnp.sum(aligned) / BS).astype(jnp.int32).reshape(1, 1)


def _sc_scatter(act, d0, d1):
    mesh = plsc.VectorSubcoreMesh(core_axis_name="c", subcore_axis_name="s")

    @functools.partial(
        pl.kernel, mesh=mesh,
        out_type=jax.ShapeDtypeStruct((MAXROWS, D), jnp.float32),
        scratch_types=[
            pltpu.VMEM((TPW, D), jnp.float32),
            pltpu.VMEM((TPW,), jnp.int32),
            pltpu.SemaphoreType.DMA,
        ],
    )
    def k(act_hbm, d0_hbm, d1_hbm, xs_hbm, rows_v, idx_v, sem):
        wid = lax.axis_index("s") * 2 + lax.axis_index("c")
        base = wid * TPW
        pltpu.sync_copy(act_hbm.at[pl.ds(base, TPW)], rows_v)
        pltpu.sync_copy(d0_hbm.at[pl.ds(base, TPW)], idx_v)
        pltpu.async_copy(rows_v, xs_hbm.at[idx_v], sem).wait()
        pltpu.sync_copy(d1_hbm.at[pl.ds(base, TPW)], idx_v)
        pltpu.async_copy(rows_v, xs_hbm.at[idx_v], sem).wait()

    return k(act, d0, d1)


def _sc_gather(y, d0, d1):
    mesh = plsc.VectorSubcoreMesh(core_axis_name="c", subcore_axis_name="s")

    @functools.partial(
        pl.kernel, mesh=mesh,
        out_type=(jax.ShapeDtypeStruct((T, D), jnp.float32),
                  jax.ShapeDtypeStruct((T, D), jnp.float32)),
        scratch_types=[
            pltpu.VMEM((TPW, D), jnp.float32),
            pltpu.VMEM((TPW,), jnp.int32),
            pltpu.SemaphoreType.DMA,
        ],
    )
    def k(y_hbm, d0_hbm, d1_hbm, g0_hbm, g1_hbm, rows_v, idx_v, sem):
        wid = lax.axis_index("s") * 2 + lax.axis_index("c")
        base = wid * TPW
        pltpu.sync_copy(d0_hbm.at[pl.ds(base, TPW)], idx_v)
        pltpu.async_copy(y_hbm.at[idx_v], rows_v, sem).wait()
        pltpu.sync_copy(rows_v, g0_hbm.at[pl.ds(base, TPW)])
        pltpu.sync_copy(d1_hbm.at[pl.ds(base, TPW)], idx_v)
        pltpu.async_copy(y_hbm.at[idx_v], rows_v, sem).wait()
        pltpu.sync_copy(rows_v, g1_hbm.at[pl.ds(base, TPW)])

    return k(y, d0, d1)


def _ffn_body(eob_ref, nb_ref, x_ref, w1_ref, w2_ref, o_ref):
    g = pl.program_id(0)

    @pl.when(g < nb_ref[0])
    def _():
        hmid = jax.nn.gelu(jnp.dot(x_ref[...], w1_ref[0],
                                   preferred_element_type=jnp.float32))
        o_ref[...] = jnp.dot(hmid, w2_ref[0],
                             preferred_element_type=jnp.float32)


def _moe_ffn(eob, nb, xs, W1, W2):
    spec = pltpu.PrefetchScalarGridSpec(
        num_scalar_prefetch=2,
        grid=(MAXBLK,),
        in_specs=[
            pl.BlockSpec((BS, D), lambda g, eob, nb: (g, 0)),
            pl.BlockSpec((1, D, HID), lambda g, eob, nb: (eob[g], 0, 0)),
            pl.BlockSpec((1, HID, D), lambda g, eob, nb: (eob[g], 0, 0)),
        ],
        out_specs=pl.BlockSpec((BS, D), lambda g, eob, nb: (g, 0)),
    )
    return pl.pallas_call(
        _ffn_body,
        grid_spec=spec,
        out_shape=jax.ShapeDtypeStruct((MAXROWS, D), jnp.float32),
    )(eob, nb, xs, W1, W2)


def _final_body(act_ref, g0_ref, g1_ref, w0_ref, w1_ref, o_ref):
    o_ref[...] = jnp.tanh(w0_ref[...] * g0_ref[...]
                          + w1_ref[...] * g1_ref[...] + act_ref[...])


def _final(act, g0, g1, w0, w1):
    return pl.pallas_call(
        _final_body,
        out_shape=jax.ShapeDtypeStruct((T, D), jnp.float32),
    )(act, g0, g1, w0, w1)


def kernel(text, img, ln_q_g, Wq, Wkv, null_k, null_v, Wo, ln_out_g, gate_W,
           expert_W1, expert_W2):
    x = text.reshape(T, D)
    im = img.reshape(SI, D)
    act, d0, d1, w0, w1, eob, nb = _proj_attention(
        x, im, ln_q_g.reshape(1, D), Wq, Wkv,
        null_k.reshape(1, DH), null_v.reshape(1, DH),
        Wo, ln_out_g.reshape(1, D), gate_W)
    d0f = d0.reshape(T)
    d1f = d1.reshape(T)
    xs = _sc_scatter(act, d0f, d1f)
    y = _moe_ffn(eob[0, :MAXBLK], nb.reshape(1), xs, expert_W1, expert_W2)
    g0, g1 = _sc_gather(y, d0f, d1f)
    out = _final(act, g0, g1, w0, w1)
    return out.reshape(1, T, D)
